# Initial kernel scaffold; baseline (speedup 1.0000x reference)
import jax
import jax.numpy as jnp
from jax.experimental import pallas as pl


def kernel(pred, target):
    raise NotImplementedError("write your pallas kernel here")


# speedup vs baseline: 2.6677x; 2.6677x over previous
import functools

import jax
import jax.numpy as jnp
from jax.experimental import pallas as pl
from jax.experimental.pallas import tpu as pltpu

CLASSES = 100000
SMOOTHING = 0.1
CONFIDENCE = 1.0 - SMOOTHING
EPS = SMOOTHING / (CLASSES - 1)
N_ROWS = 1024
C_BLK = 2048
N_BLK = (CLASSES + C_BLK - 1) // C_BLK


def _loss_kernel(target_ref, pred_ref, out_ref, acc_ref):
    i = pl.program_id(0)
    x = pred_ref[...]
    cols = jax.lax.broadcasted_iota(jnp.int32, x.shape, 1) + i * C_BLK

    def accumulate(partial):
        @pl.when(i == 0)
        def _init():
            acc_ref[...] = partial

        @pl.when(i > 0)
        def _accum():
            acc_ref[...] += partial

    def partials(xe, xs):
        sumexp = jnp.sum(jnp.exp(xe), axis=1, keepdims=True)
        sumpred = jnp.sum(xs, axis=1, keepdims=True)
        tgt = jnp.sum(jnp.where(cols == target_ref[...], xs, 0.0),
                      axis=1, keepdims=True)
        return jnp.concatenate([sumexp, sumpred, tgt], axis=1)

    @pl.when(i < N_BLK - 1)
    def _full_block():
        accumulate(partials(x, x))

    @pl.when(i == N_BLK - 1)
    def _tail_block():
        valid = cols < CLASSES
        accumulate(partials(jnp.where(valid, x, -jnp.inf),
                            jnp.where(valid, x, 0.0)))

    @pl.when(i == N_BLK - 1)
    def _finalize():
        acc = acc_ref[...]
        lse = jnp.log(acc[:, 0:1])
        per_row = lse - EPS * acc[:, 1:2] - (CONFIDENCE - EPS) * acc[:, 2:3]
        out_ref[0, 0] = jnp.sum(per_row) / N_ROWS


@functools.partial(jax.jit, static_argnames=())
def _run(pred, target):
    target2d = target.astype(jnp.int32).reshape(N_ROWS, 1)
    out = pl.pallas_call(
        _loss_kernel,
        grid=(N_BLK,),
        in_specs=[
            pl.BlockSpec((N_ROWS, 1), lambda i: (0, 0)),
            pl.BlockSpec((N_ROWS, C_BLK), lambda i: (0, i)),
        ],
        out_specs=pl.BlockSpec((1, 1), lambda i: (0, 0),
                               memory_space=pltpu.SMEM),
        out_shape=jax.ShapeDtypeStruct((1, 1), jnp.float32),
        scratch_shapes=[pltpu.VMEM((N_ROWS, 3), jnp.float32)],
        compiler_params=pltpu.CompilerParams(
            dimension_semantics=("arbitrary",),
        ),
    )(target2d, pred)
    return out[0, 0]


def kernel(pred, target):
    return _run(pred, target)
